# Initial kernel scaffold; baseline (speedup 1.0000x reference)
import jax
import jax.numpy as jnp
from jax.experimental import pallas as pl


def kernel(predictions, bboxes):
    raise NotImplementedError("write your pallas kernel here")


# speedup vs baseline: 1.0663x; 1.0663x over previous
import math

import jax
import jax.numpy as jnp
import numpy as np
from jax.experimental import pallas as pl
from jax.experimental.pallas import tpu as pltpu

_B, _H, _W, _A, _C = 32, 56, 56, 9, 6
_N = 20
_LANES = 128
_ROWS = _H * _W * _A * _C // _LANES
_NF = 10


def _anchor_wh():
    a = []
    for s in (32, 64, 128):
        for r in (0.5, 1.0, 2.0):
            a.append([s * math.sqrt(r) / 224.0, s / math.sqrt(r) / 224.0])
    return jnp.asarray(a, dtype=jnp.float32)


def _loss_kernel(rows_smem, pred_ref, bbl_ref, bbc_ref, out_ref, scratch):
    b = pl.program_id(0)

    x = pred_ref[0]
    r_iota = jax.lax.broadcasted_iota(jnp.int32, (_ROWS, _LANES), 0)
    c_iota = jax.lax.broadcasted_iota(jnp.int32, (_ROWS, _LANES), 1)
    ch4 = ((2 * r_iota + c_iota) % _C) == 4
    p_all = jax.nn.sigmoid(x)
    bce0 = -jnp.maximum(jnp.log(1.0 - p_all), -100.0)
    s0 = jnp.sum(jnp.where(ch4, bce0, 0.0))

    bbc = bbc_ref[0]
    cx, cy = bbc[:, 0:1], bbc[:, 1:2]
    w, h = bbc[:, 2:3], bbc[:, 3:4]
    gxf, gyf = bbc[:, 4:5], bbc[:, 5:6]
    flat_c = bbc[:, 6:7].astype(jnp.int32)
    baw, bah = bbc[:, 7:8], bbc[:, 8:9]
    valid_c = bbc[:, 9:10] != 0.0
    tx = cx * _W - gxf
    ty = cy * _H - gyf
    tw = jnp.log(w / baw + 1e-16)
    th = jnp.log(h / bah + 1e-16)

    bbl = bbl_ref[0]
    flat_r = bbl[6:7, :].astype(jnp.int32)
    valid_r = bbl[9:10, :] != 0.0

    n_c = jax.lax.broadcasted_iota(jnp.int32, (_N, 1), 0)
    n_r = jax.lax.broadcasted_iota(jnp.int32, (1, _N), 1)
    id_c = jnp.where(valid_c, flat_c, -1 - n_c)
    id_r = jnp.where(valid_r, flat_r, -1 - n_r)
    killed = jnp.any((id_c == id_r) & (n_r > n_c), axis=1, keepdims=True)
    alive = (valid_c & ~killed).astype(jnp.float32)

    row_start = jnp.minimum(flat_c // _LANES, _ROWS - 2)
    lane_off = flat_c - _LANES * row_start
    for n in range(_N):
        row = jnp.minimum(rows_smem[b, n], _ROWS - 2)
        pair = pred_ref[0, pl.ds(row, 2), :]
        scratch[n : n + 1, 0:_LANES] = pair[0:1, :]
        scratch[n : n + 1, _LANES : 2 * _LANES] = pair[1:2, :]

    y = scratch[...]
    rel = jax.lax.broadcasted_iota(jnp.int32, (_N, 2 * _LANES), 1) - lane_off
    tgrid = (
        jnp.where(rel == 0, tx, 0.0)
        + jnp.where(rel == 1, ty, 0.0)
        + jnp.where(rel == 2, tw, 0.0)
        + jnp.where(rel == 3, th, 0.0)
    )
    in4 = (rel >= 0) & (rel < 4)
    coord_n = jnp.sum(jnp.where(in4, (y - tgrid) ** 2, 0.0), axis=1, keepdims=True)
    v4 = jnp.sum(jnp.where(rel == 4, y, 0.0), axis=1, keepdims=True)
    pv = jax.nn.sigmoid(v4)
    obj_n = -jnp.maximum(jnp.log(pv), -100.0)
    ncorr_n = -jnp.maximum(jnp.log(1.0 - pv), -100.0)

    npos_b = jnp.sum(alive)
    sobj_b = jnp.sum(alive * obj_n)
    sncorr_b = jnp.sum(alive * ncorr_n)
    scoord_b = jnp.sum(alive * coord_n)

    lane_o = jax.lax.broadcasted_iota(jnp.int32, (1, _LANES), 1)
    delta = (
        jnp.where(lane_o == 0, s0, 0.0)
        + jnp.where(lane_o == 1, npos_b, 0.0)
        + jnp.where(lane_o == 2, sobj_b, 0.0)
        + jnp.where(lane_o == 3, sncorr_b, 0.0)
        + jnp.where(lane_o == 4, scoord_b, 0.0)
    )

    @pl.when(b == 0)
    def _zero():
        out_ref[...] = jnp.zeros_like(out_ref)

    out_ref[...] = out_ref[...] + delta


def kernel(predictions, bboxes):
    B, H, W, A, C = predictions.shape
    f32 = jnp.float32
    pred3 = predictions.reshape(B, _ROWS, _LANES)

    anchors = _anchor_wh()
    cx, cy = bboxes[..., 0], bboxes[..., 1]
    w, h = bboxes[..., 2], bboxes[..., 3]
    valid = ~jnp.all(bboxes == 0.0, axis=-1)
    gx = jnp.clip(jnp.floor(cx * W).astype(jnp.int32), 0, W - 1)
    gy = jnp.clip(jnp.floor(cy * H).astype(jnp.int32), 0, H - 1)
    aw = anchors[:, 0][None, None, :]
    ah = anchors[:, 1][None, None, :]
    inter = jnp.minimum(w[..., None], aw) * jnp.minimum(h[..., None], ah)
    union = (w * h)[..., None] + aw * ah - inter
    iou = inter / (union + 1e-16)
    best = jnp.argmax(iou, axis=-1)
    baw = anchors[best, 0]
    bah = anchors[best, 1]
    flat = ((gy * W + gx) * A + best.astype(jnp.int32)) * C
    rows = jnp.minimum(flat // _LANES, _ROWS - 2)

    feat = jnp.stack(
        [
            cx,
            cy,
            w,
            h,
            gx.astype(f32),
            gy.astype(f32),
            flat.astype(f32),
            baw,
            bah,
            valid.astype(f32),
        ],
        axis=-1,
    )
    feat_t = jnp.transpose(feat, (0, 2, 1))

    grid_spec = pltpu.PrefetchScalarGridSpec(
        num_scalar_prefetch=1,
        grid=(B,),
        in_specs=[
            pl.BlockSpec((1, _ROWS, _LANES), lambda b, s: (b, 0, 0)),
            pl.BlockSpec((1, _NF, _N), lambda b, s: (b, 0, 0)),
            pl.BlockSpec((1, _N, _NF), lambda b, s: (b, 0, 0)),
        ],
        out_specs=pl.BlockSpec((1, _LANES), lambda b, s: (0, 0)),
        scratch_shapes=[pltpu.VMEM((_N, 2 * _LANES), jnp.float32)],
    )
    acc = pl.pallas_call(
        _loss_kernel,
        grid_spec=grid_spec,
        out_shape=jax.ShapeDtypeStruct((1, _LANES), jnp.float32),
    )(rows, pred3, feat_t, feat)

    s0, n_pos = acc[0, 0], acc[0, 1]
    sobj, sncorr, scoord = acc[0, 2], acc[0, 3], acc[0, 4]
    n_neg = jnp.float32(B * H * W * A) - n_pos
    coord_loss = 5.0 * scoord
    obj_loss = 1.0 * sobj
    noobj_loss = 0.5 * (s0 - sncorr)
    coord_loss = jnp.where(n_pos > 0, coord_loss / n_pos, coord_loss)
    obj_loss = jnp.where(n_pos > 0, obj_loss / n_pos, obj_loss)
    noobj_loss = jnp.where(n_neg > 0, noobj_loss / n_neg, noobj_loss)
    cls_loss = jnp.asarray(0.0, jnp.float32)
    total_loss = coord_loss + obj_loss + noobj_loss + cls_loss
    return (total_loss, coord_loss, obj_loss, noobj_loss, cls_loss)


# speedup vs baseline: 4.1175x; 3.8613x over previous
import math

import jax
import jax.numpy as jnp
from jax.experimental import pallas as pl
from jax.experimental.pallas import tpu as pltpu

_B, _H, _W, _A, _C = 32, 56, 56, 9, 6
_N = 20
_NF = 10


def _anchor_wh():
    a = []
    for s in (32, 64, 128):
        for r in (0.5, 1.0, 2.0):
            a.append([s * math.sqrt(r) / 224.0, s / math.sqrt(r) / 224.0])
    return jnp.asarray(a, dtype=jnp.float32)


def _loss_kernel(idx_smem, pred_ref, bbl_ref, bbc_ref, out_ref, scratch):
    b = pl.program_id(0)

    x = pred_ref[0, :, 4, :, :]
    p_all = jax.nn.sigmoid(x)
    s0 = jnp.sum(-jnp.maximum(jnp.log(1.0 - p_all), -100.0))

    bbc = bbc_ref[0]
    cx, cy = bbc[:, 0:1], bbc[:, 1:2]
    w, h = bbc[:, 2:3], bbc[:, 3:4]
    gxf, gyf = bbc[:, 4:5], bbc[:, 5:6]
    flat_c = bbc[:, 6:7].astype(jnp.int32)
    baw, bah = bbc[:, 7:8], bbc[:, 8:9]
    valid_c = bbc[:, 9:10] != 0.0
    tx = cx * _W - gxf
    ty = cy * _H - gyf
    tw = jnp.log(w / baw + 1e-16)
    th = jnp.log(h / bah + 1e-16)
    gx_i = gxf.astype(jnp.int32)

    bbl = bbl_ref[0]
    flat_r = bbl[6:7, :].astype(jnp.int32)
    valid_r = bbl[9:10, :] != 0.0

    n_c = jax.lax.broadcasted_iota(jnp.int32, (_N, 1), 0)
    n_r = jax.lax.broadcasted_iota(jnp.int32, (1, _N), 1)
    id_c = jnp.where(valid_c, flat_c, -1 - n_c)
    id_r = jnp.where(valid_r, flat_r, -1 - n_r)
    killed = jnp.any((id_c == id_r) & (n_r > n_c), axis=1, keepdims=True)
    alive = (valid_c & ~killed).astype(jnp.float32)

    for n in range(_N):
        bestn = idx_smem[b, 0, n]
        gyn = idx_smem[b, 1, n]
        scratch[n] = pred_ref[0, bestn, 0:5, gyn, :]

    lane = jax.lax.broadcasted_iota(jnp.int32, (_N, _W), 1)
    sel = lane == gx_i
    v = [
        jnp.sum(jnp.where(sel, scratch[:, c, :], 0.0), axis=1, keepdims=True)
        for c in range(5)
    ]
    coord_n = (v[0] - tx) ** 2 + (v[1] - ty) ** 2 + (v[2] - tw) ** 2 + (v[3] - th) ** 2
    pv = jax.nn.sigmoid(v[4])
    obj_n = -jnp.maximum(jnp.log(pv), -100.0)
    ncorr_n = -jnp.maximum(jnp.log(1.0 - pv), -100.0)

    npos_b = jnp.sum(alive)
    sobj_b = jnp.sum(alive * obj_n)
    sncorr_b = jnp.sum(alive * ncorr_n)
    scoord_b = jnp.sum(alive * coord_n)

    lane_o = jax.lax.broadcasted_iota(jnp.int32, (1, 128), 1)
    delta = (
        jnp.where(lane_o == 0, s0, 0.0)
        + jnp.where(lane_o == 1, npos_b, 0.0)
        + jnp.where(lane_o == 2, sobj_b, 0.0)
        + jnp.where(lane_o == 3, sncorr_b, 0.0)
        + jnp.where(lane_o == 4, scoord_b, 0.0)
    )

    @pl.when(b == 0)
    def _zero():
        out_ref[...] = jnp.zeros_like(out_ref)

    out_ref[...] = out_ref[...] + delta


def kernel(predictions, bboxes):
    B, H, W, A, C = predictions.shape
    f32 = jnp.float32
    pred_t = jnp.transpose(predictions, (0, 3, 4, 1, 2))

    anchors = _anchor_wh()
    cx, cy = bboxes[..., 0], bboxes[..., 1]
    w, h = bboxes[..., 2], bboxes[..., 3]
    valid = ~jnp.all(bboxes == 0.0, axis=-1)
    gx = jnp.clip(jnp.floor(cx * W).astype(jnp.int32), 0, W - 1)
    gy = jnp.clip(jnp.floor(cy * H).astype(jnp.int32), 0, H - 1)
    aw = anchors[:, 0][None, None, :]
    ah = anchors[:, 1][None, None, :]
    inter = jnp.minimum(w[..., None], aw) * jnp.minimum(h[..., None], ah)
    union = (w * h)[..., None] + aw * ah - inter
    iou = inter / (union + 1e-16)
    best = jnp.argmax(iou, axis=-1).astype(jnp.int32)
    baw = anchors[best, 0]
    bah = anchors[best, 1]
    flat = (gy * W + gx) * A + best

    idx = jnp.stack([best, gy], axis=1)
    feat = jnp.stack(
        [
            cx,
            cy,
            w,
            h,
            gx.astype(f32),
            gy.astype(f32),
            flat.astype(f32),
            baw,
            bah,
            valid.astype(f32),
        ],
        axis=-1,
    )
    feat_t = jnp.transpose(feat, (0, 2, 1))

    grid_spec = pltpu.PrefetchScalarGridSpec(
        num_scalar_prefetch=1,
        grid=(B,),
        in_specs=[
            pl.BlockSpec((1, A, C, H, W), lambda b, s: (b, 0, 0, 0, 0)),
            pl.BlockSpec((1, _NF, _N), lambda b, s: (b, 0, 0)),
            pl.BlockSpec((1, _N, _NF), lambda b, s: (b, 0, 0)),
        ],
        out_specs=pl.BlockSpec((1, 128), lambda b, s: (0, 0)),
        scratch_shapes=[pltpu.VMEM((_N, 5, _W), jnp.float32)],
    )
    acc = pl.pallas_call(
        _loss_kernel,
        grid_spec=grid_spec,
        out_shape=jax.ShapeDtypeStruct((1, 128), jnp.float32),
    )(idx, pred_t, feat_t, feat)

    s0, n_pos = acc[0, 0], acc[0, 1]
    sobj, sncorr, scoord = acc[0, 2], acc[0, 3], acc[0, 4]
    n_neg = jnp.float32(B * H * W * A) - n_pos
    coord_loss = 5.0 * scoord
    obj_loss = 1.0 * sobj
    noobj_loss = 0.5 * (s0 - sncorr)
    coord_loss = jnp.where(n_pos > 0, coord_loss / n_pos, coord_loss)
    obj_loss = jnp.where(n_pos > 0, obj_loss / n_pos, obj_loss)
    noobj_loss = jnp.where(n_neg > 0, noobj_loss / n_neg, noobj_loss)
    cls_loss = jnp.asarray(0.0, jnp.float32)
    total_loss = coord_loss + obj_loss + noobj_loss + cls_loss
    return (total_loss, coord_loss, obj_loss, noobj_loss, cls_loss)


# speedup vs baseline: 5.2191x; 1.2676x over previous
import math

import jax
import jax.numpy as jnp
from jax.experimental import pallas as pl
from jax.experimental.pallas import tpu as pltpu

_B, _H, _W, _A, _C = 32, 56, 56, 9, 6
_N = 20
_ANCHORS = [
    (s * math.sqrt(r) / 224.0, s / math.sqrt(r) / 224.0)
    for s in (32, 64, 128)
    for r in (0.5, 1.0, 2.0)
]


def _issue_batch(idx_smem, pred_any, vg, sems, bi):
    for n in range(_N):
        bestn = idx_smem[bi, 0, n]
        gyn = idx_smem[bi, 1, n]
        pltpu.make_async_copy(
            pred_any.at[bi, bestn, pl.ds(0, 5), gyn, :],
            vg.at[bi, :, n, :],
            sems.at[bi],
        ).start()


def _wait_batch(idx_smem, pred_any, vg, sems, bi):
    for n in range(_N):
        bestn = idx_smem[bi, 0, n]
        gyn = idx_smem[bi, 1, n]
        pltpu.make_async_copy(
            pred_any.at[bi, bestn, pl.ds(0, 5), gyn, :],
            vg.at[bi, :, n, :],
            sems.at[bi],
        ).wait()


def _loss_kernel(idx_smem, pred_ref, pred_any, bbc_ref, ixc_ref, ixr_ref, out_ref, vg, sems):
    b = pl.program_id(0)
    f32 = jnp.float32

    @pl.when(b == 0)
    def _prime():
        _issue_batch(idx_smem, pred_any, vg, sems, 0)
        _issue_batch(idx_smem, pred_any, vg, sems, 1)

    @pl.when(b < _B - 2)
    def _ahead():
        _issue_batch(idx_smem, pred_any, vg, sems, b + 2)

    x = pred_ref[0, :, 0, :, :]
    p_all = jax.nn.sigmoid(x)
    s0 = jnp.sum(-jnp.maximum(jnp.log(1.0 - p_all), -100.0))

    bbc = bbc_ref[0]
    cx, cy = bbc[:, 0:1], bbc[:, 1:2]
    w, h = bbc[:, 2:3], bbc[:, 3:4]
    ixc = ixc_ref[0]
    best_c, gx_c, gy_c = ixc[:, 0:1], ixc[:, 1:2], ixc[:, 2:3]
    valid_c = ixc[:, 3:4] != 0
    tx = cx * _W - gx_c.astype(f32)
    ty = cy * _H - gy_c.astype(f32)
    baw = jnp.zeros(best_c.shape, f32)
    bah = jnp.zeros(best_c.shape, f32)
    for k, (awk, ahk) in enumerate(_ANCHORS):
        baw = jnp.where(best_c == k, awk, baw)
        bah = jnp.where(best_c == k, ahk, bah)
    tw = jnp.log(w / baw + 1e-16)
    th = jnp.log(h / bah + 1e-16)
    flat_c = (gy_c * _W + gx_c) * _A + best_c

    ixr = ixr_ref[0]
    flat_r = (ixr[2:3, :] * _W + ixr[1:2, :]) * _A + ixr[0:1, :]
    valid_r = ixr[3:4, :] != 0

    n_c = jax.lax.broadcasted_iota(jnp.int32, (_N, 1), 0)
    n_r = jax.lax.broadcasted_iota(jnp.int32, (1, _N), 1)
    id_c = jnp.where(valid_c, flat_c, -1 - n_c)
    id_r = jnp.where(valid_r, flat_r, -1 - n_r)
    killed = jnp.any((id_c == id_r) & (n_r > n_c), axis=1, keepdims=True)
    alive = (valid_c & ~killed).astype(f32)

    _wait_batch(idx_smem, pred_any, vg, sems, b)
    lane = jax.lax.broadcasted_iota(jnp.int32, (_N, _W), 1)
    sel = lane == gx_c
    v = [
        jnp.sum(jnp.where(sel, vg[b, c, :, :], 0.0), axis=1, keepdims=True)
        for c in range(5)
    ]
    coord_n = (v[0] - tx) ** 2 + (v[1] - ty) ** 2 + (v[2] - tw) ** 2 + (v[3] - th) ** 2
    pv = jax.nn.sigmoid(v[4])
    obj_n = -jnp.maximum(jnp.log(pv), -100.0)
    ncorr_n = -jnp.maximum(jnp.log(1.0 - pv), -100.0)

    npos_b = jnp.sum(alive)
    sobj_b = jnp.sum(alive * obj_n)
    sncorr_b = jnp.sum(alive * ncorr_n)
    scoord_b = jnp.sum(alive * coord_n)

    lane_o = jax.lax.broadcasted_iota(jnp.int32, (1, 128), 1)
    delta = (
        jnp.where(lane_o == 0, s0, 0.0)
        + jnp.where(lane_o == 1, npos_b, 0.0)
        + jnp.where(lane_o == 2, sobj_b, 0.0)
        + jnp.where(lane_o == 3, sncorr_b, 0.0)
        + jnp.where(lane_o == 4, scoord_b, 0.0)
    )

    @pl.when(b == 0)
    def _zero():
        out_ref[...] = jnp.zeros_like(out_ref)

    out_ref[...] = out_ref[...] + delta


def kernel(predictions, bboxes):
    B, H, W, A, C = predictions.shape
    pred_t = jnp.transpose(predictions, (0, 3, 4, 1, 2))

    anchors = jnp.asarray(_ANCHORS, dtype=jnp.float32)
    cx, cy = bboxes[..., 0], bboxes[..., 1]
    w, h = bboxes[..., 2], bboxes[..., 3]
    valid = ~jnp.all(bboxes == 0.0, axis=-1)
    gx = jnp.clip(jnp.floor(cx * W).astype(jnp.int32), 0, W - 1)
    gy = jnp.clip(jnp.floor(cy * H).astype(jnp.int32), 0, H - 1)
    aw = anchors[:, 0][None, None, :]
    ah = anchors[:, 1][None, None, :]
    inter = jnp.minimum(w[..., None], aw) * jnp.minimum(h[..., None], ah)
    union = (w * h)[..., None] + aw * ah - inter
    iou = inter / (union + 1e-16)
    best = jnp.argmax(iou, axis=-1).astype(jnp.int32)

    idx = jnp.stack([best, gy], axis=1)
    ixc = jnp.stack([best, gx, gy, valid.astype(jnp.int32)], axis=-1)
    ixr = jnp.transpose(ixc, (0, 2, 1))

    grid_spec = pltpu.PrefetchScalarGridSpec(
        num_scalar_prefetch=1,
        grid=(B,),
        in_specs=[
            pl.BlockSpec((1, A, 1, H, W), lambda b, s: (b, 0, 4, 0, 0)),
            pl.BlockSpec(memory_space=pltpu.MemorySpace.HBM),
            pl.BlockSpec((1, _N, 4), lambda b, s: (b, 0, 0)),
            pl.BlockSpec((1, _N, 4), lambda b, s: (b, 0, 0)),
            pl.BlockSpec((1, 4, _N), lambda b, s: (b, 0, 0)),
        ],
        out_specs=pl.BlockSpec((1, 128), lambda b, s: (0, 0)),
        scratch_shapes=[
            pltpu.VMEM((_B, 5, _N, _W), jnp.float32),
            pltpu.SemaphoreType.DMA((_B,)),
        ],
    )
    acc = pl.pallas_call(
        _loss_kernel,
        grid_spec=grid_spec,
        out_shape=jax.ShapeDtypeStruct((1, 128), jnp.float32),
    )(idx, pred_t, pred_t, bboxes, ixc, ixr)

    s0, n_pos = acc[0, 0], acc[0, 1]
    sobj, sncorr, scoord = acc[0, 2], acc[0, 3], acc[0, 4]
    n_neg = jnp.float32(B * H * W * A) - n_pos
    coord_loss = 5.0 * scoord
    obj_loss = 1.0 * sobj
    noobj_loss = 0.5 * (s0 - sncorr)
    coord_loss = jnp.where(n_pos > 0, coord_loss / n_pos, coord_loss)
    obj_loss = jnp.where(n_pos > 0, obj_loss / n_pos, obj_loss)
    noobj_loss = jnp.where(n_neg > 0, noobj_loss / n_neg, noobj_loss)
    cls_loss = jnp.asarray(0.0, jnp.float32)
    total_loss = coord_loss + obj_loss + noobj_loss + cls_loss
    return (total_loss, coord_loss, obj_loss, noobj_loss, cls_loss)


# speedup vs baseline: 5.9115x; 1.1326x over previous
import math

import jax
import jax.numpy as jnp
from jax.experimental import pallas as pl
from jax.experimental.pallas import tpu as pltpu

_B, _H, _W, _A, _C = 32, 56, 56, 9, 6
_N = 20
_NB = 8
_ANCHORS = [
    (s * math.sqrt(r) / 224.0, s / math.sqrt(r) / 224.0)
    for s in (32, 64, 128)
    for r in (0.5, 1.0, 2.0)
]


def _issue_batch(idx_smem, pred_any, vg, sems, bi):
    for n in range(_N):
        bestn = idx_smem[bi, 0, n]
        gyn = idx_smem[bi, 1, n]
        pltpu.make_async_copy(
            pred_any.at[bi, bestn, pl.ds(0, 5), gyn, :],
            vg.at[bi, :, n, :],
            sems.at[bi],
        ).start()


def _wait_batch(idx_smem, pred_any, vg, sems, bi):
    for n in range(_N):
        bestn = idx_smem[bi, 0, n]
        gyn = idx_smem[bi, 1, n]
        pltpu.make_async_copy(
            pred_any.at[bi, bestn, pl.ds(0, 5), gyn, :],
            vg.at[bi, :, n, :],
            sems.at[bi],
        ).wait()


def _loss_kernel(idx_smem, pred_ref, pred_any, bbc_ref, ixc_ref, ixr_ref, out_ref, vg, sems):
    g = pl.program_id(0)
    f32 = jnp.float32

    @pl.when(g == 0)
    def _prime():
        for i in range(_NB):
            _issue_batch(idx_smem, pred_any, vg, sems, i)

    @pl.when(g < _B // _NB - 1)
    def _ahead():
        for i in range(_NB):
            _issue_batch(idx_smem, pred_any, vg, sems, (g + 1) * _NB + i)

    x = pred_ref[:, :, 0, :, :]
    p_all = jax.nn.sigmoid(x)
    s0 = jnp.sum(-jnp.maximum(jnp.log(1.0 - p_all), -100.0))

    npos_b = jnp.float32(0.0)
    sobj_b = jnp.float32(0.0)
    sncorr_b = jnp.float32(0.0)
    scoord_b = jnp.float32(0.0)
    n_c = jax.lax.broadcasted_iota(jnp.int32, (_N, 1), 0)
    n_r = jax.lax.broadcasted_iota(jnp.int32, (1, _N), 1)
    lane = jax.lax.broadcasted_iota(jnp.int32, (_N, _W), 1)

    for i in range(_NB):
        b = g * _NB + i
        bbc = bbc_ref[b]
        cx, cy = bbc[:, 0:1], bbc[:, 1:2]
        w, h = bbc[:, 2:3], bbc[:, 3:4]
        ixc = ixc_ref[b]
        best_c, gx_c, gy_c = ixc[:, 0:1], ixc[:, 1:2], ixc[:, 2:3]
        valid_c = ixc[:, 3:4] != 0
        tx = cx * _W - gx_c.astype(f32)
        ty = cy * _H - gy_c.astype(f32)
        baw = jnp.zeros(best_c.shape, f32)
        bah = jnp.zeros(best_c.shape, f32)
        for k, (awk, ahk) in enumerate(_ANCHORS):
            baw = jnp.where(best_c == k, awk, baw)
            bah = jnp.where(best_c == k, ahk, bah)
        tw = jnp.log(w / baw + 1e-16)
        th = jnp.log(h / bah + 1e-16)
        flat_c = (gy_c * _W + gx_c) * _A + best_c

        ixr = ixr_ref[b]
        flat_r = (ixr[2:3, :] * _W + ixr[1:2, :]) * _A + ixr[0:1, :]
        valid_r = ixr[3:4, :] != 0

        id_c = jnp.where(valid_c, flat_c, -1 - n_c)
        id_r = jnp.where(valid_r, flat_r, -1 - n_r)
        killed = jnp.any((id_c == id_r) & (n_r > n_c), axis=1, keepdims=True)
        alive = (valid_c & ~killed).astype(f32)

        _wait_batch(idx_smem, pred_any, vg, sems, b)
        sel = lane == gx_c
        v = [
            jnp.sum(jnp.where(sel, vg[b, c, :, :], 0.0), axis=1, keepdims=True)
            for c in range(5)
        ]
        coord_n = (
            (v[0] - tx) ** 2 + (v[1] - ty) ** 2 + (v[2] - tw) ** 2 + (v[3] - th) ** 2
        )
        pv = jax.nn.sigmoid(v[4])
        obj_n = -jnp.maximum(jnp.log(pv), -100.0)
        ncorr_n = -jnp.maximum(jnp.log(1.0 - pv), -100.0)

        npos_b += jnp.sum(alive)
        sobj_b += jnp.sum(alive * obj_n)
        sncorr_b += jnp.sum(alive * ncorr_n)
        scoord_b += jnp.sum(alive * coord_n)

    lane_o = jax.lax.broadcasted_iota(jnp.int32, (1, 128), 1)
    delta = (
        jnp.where(lane_o == 0, s0, 0.0)
        + jnp.where(lane_o == 1, npos_b, 0.0)
        + jnp.where(lane_o == 2, sobj_b, 0.0)
        + jnp.where(lane_o == 3, sncorr_b, 0.0)
        + jnp.where(lane_o == 4, scoord_b, 0.0)
    )

    @pl.when(g == 0)
    def _zero():
        out_ref[...] = jnp.zeros_like(out_ref)

    out_ref[...] = out_ref[...] + delta


def kernel(predictions, bboxes):
    B, H, W, A, C = predictions.shape
    pred_t = jnp.transpose(predictions, (0, 3, 4, 1, 2))

    anchors = jnp.asarray(_ANCHORS, dtype=jnp.float32)
    cx, cy = bboxes[..., 0], bboxes[..., 1]
    w, h = bboxes[..., 2], bboxes[..., 3]
    valid = ~jnp.all(bboxes == 0.0, axis=-1)
    gx = jnp.clip(jnp.floor(cx * W).astype(jnp.int32), 0, W - 1)
    gy = jnp.clip(jnp.floor(cy * H).astype(jnp.int32), 0, H - 1)
    aw = anchors[:, 0][None, None, :]
    ah = anchors[:, 1][None, None, :]
    inter = jnp.minimum(w[..., None], aw) * jnp.minimum(h[..., None], ah)
    union = (w * h)[..., None] + aw * ah - inter
    iou = inter / (union + 1e-16)
    best = jnp.argmax(iou, axis=-1).astype(jnp.int32)

    idx = jnp.stack([best, gy], axis=1)
    ixc = jnp.stack([best, gx, gy, valid.astype(jnp.int32)], axis=-1)
    ixr = jnp.transpose(ixc, (0, 2, 1))

    grid_spec = pltpu.PrefetchScalarGridSpec(
        num_scalar_prefetch=1,
        grid=(B // _NB,),
        in_specs=[
            pl.BlockSpec((_NB, A, 1, H, W), lambda b, s: (b, 0, 4, 0, 0)),
            pl.BlockSpec(memory_space=pltpu.MemorySpace.HBM),
            pl.BlockSpec((B, _N, 4), lambda b, s: (0, 0, 0)),
            pl.BlockSpec((B, _N, 4), lambda b, s: (0, 0, 0)),
            pl.BlockSpec((B, 4, _N), lambda b, s: (0, 0, 0)),
        ],
        out_specs=pl.BlockSpec((1, 128), lambda b, s: (0, 0)),
        scratch_shapes=[
            pltpu.VMEM((_B, 5, _N, _W), jnp.float32),
            pltpu.SemaphoreType.DMA((_B,)),
        ],
    )
    acc = pl.pallas_call(
        _loss_kernel,
        grid_spec=grid_spec,
        out_shape=jax.ShapeDtypeStruct((1, 128), jnp.float32),
    )(idx, pred_t, pred_t, bboxes, ixc, ixr)

    s0, n_pos = acc[0, 0], acc[0, 1]
    sobj, sncorr, scoord = acc[0, 2], acc[0, 3], acc[0, 4]
    n_neg = jnp.float32(B * H * W * A) - n_pos
    coord_loss = 5.0 * scoord
    obj_loss = 1.0 * sobj
    noobj_loss = 0.5 * (s0 - sncorr)
    coord_loss = jnp.where(n_pos > 0, coord_loss / n_pos, coord_loss)
    obj_loss = jnp.where(n_pos > 0, obj_loss / n_pos, obj_loss)
    noobj_loss = jnp.where(n_neg > 0, noobj_loss / n_neg, noobj_loss)
    cls_loss = jnp.asarray(0.0, jnp.float32)
    total_loss = coord_loss + obj_loss + noobj_loss + cls_loss
    return (total_loss, coord_loss, obj_loss, noobj_loss, cls_loss)


# speedup vs baseline: 6.0743x; 1.0276x over previous
import math

import jax
import jax.numpy as jnp
from jax.experimental import pallas as pl
from jax.experimental.pallas import tpu as pltpu

_B, _H, _W, _A, _C = 32, 56, 56, 9, 6
_N = 20
_NB = 8
_ANCHORS = [
    (s * math.sqrt(r) / 224.0, s / math.sqrt(r) / 224.0)
    for s in (32, 64, 128)
    for r in (0.5, 1.0, 2.0)
]


def _loss_kernel(idx_smem, pred_ref, bbc_ref, ixc_ref, ixr_ref, out_ref, scr):
    g = pl.program_id(0)
    f32 = jnp.float32

    x = pred_ref[:, :, 4, :, :]
    p_all = jax.nn.sigmoid(x)
    s0 = jnp.sum(-jnp.maximum(jnp.log(1.0 - p_all), -100.0))

    npos_b = jnp.float32(0.0)
    sobj_b = jnp.float32(0.0)
    sncorr_b = jnp.float32(0.0)
    scoord_b = jnp.float32(0.0)
    n_c = jax.lax.broadcasted_iota(jnp.int32, (_N, 1), 0)
    n_r = jax.lax.broadcasted_iota(jnp.int32, (1, _N), 1)
    lane = jax.lax.broadcasted_iota(jnp.int32, (_N, _W), 1)

    for i in range(_NB):
        b = g * _NB + i
        bbc = bbc_ref[b]
        cx, cy = bbc[:, 0:1], bbc[:, 1:2]
        w, h = bbc[:, 2:3], bbc[:, 3:4]
        ixc = ixc_ref[b]
        best_c, gx_c, gy_c = ixc[:, 0:1], ixc[:, 1:2], ixc[:, 2:3]
        valid_c = ixc[:, 3:4] != 0
        tx = cx * _W - gx_c.astype(f32)
        ty = cy * _H - gy_c.astype(f32)
        baw = jnp.zeros(best_c.shape, f32)
        bah = jnp.zeros(best_c.shape, f32)
        for k, (awk, ahk) in enumerate(_ANCHORS):
            baw = jnp.where(best_c == k, awk, baw)
            bah = jnp.where(best_c == k, ahk, bah)
        tw = jnp.log(w / baw + 1e-16)
        th = jnp.log(h / bah + 1e-16)
        flat_c = (gy_c * _W + gx_c) * _A + best_c

        ixr = ixr_ref[b]
        flat_r = (ixr[2:3, :] * _W + ixr[1:2, :]) * _A + ixr[0:1, :]
        valid_r = ixr[3:4, :] != 0

        id_c = jnp.where(valid_c, flat_c, -1 - n_c)
        id_r = jnp.where(valid_r, flat_r, -1 - n_r)
        killed = jnp.any((id_c == id_r) & (n_r > n_c), axis=1, keepdims=True)
        alive = (valid_c & ~killed).astype(f32)

        for n in range(_N):
            bestn = idx_smem[b, 0, n]
            gyn = idx_smem[b, 1, n]
            scr[n] = pred_ref[i, bestn, 0:5, gyn, :]
        sel = lane == gx_c
        v = [
            jnp.sum(jnp.where(sel, scr[:, c, :], 0.0), axis=1, keepdims=True)
            for c in range(5)
        ]
        coord_n = (
            (v[0] - tx) ** 2 + (v[1] - ty) ** 2 + (v[2] - tw) ** 2 + (v[3] - th) ** 2
        )
        pv = jax.nn.sigmoid(v[4])
        obj_n = -jnp.maximum(jnp.log(pv), -100.0)
        ncorr_n = -jnp.maximum(jnp.log(1.0 - pv), -100.0)

        npos_b += jnp.sum(alive)
        sobj_b += jnp.sum(alive * obj_n)
        sncorr_b += jnp.sum(alive * ncorr_n)
        scoord_b += jnp.sum(alive * coord_n)

    lane_o = jax.lax.broadcasted_iota(jnp.int32, (1, 128), 1)
    delta = (
        jnp.where(lane_o == 0, s0, 0.0)
        + jnp.where(lane_o == 1, npos_b, 0.0)
        + jnp.where(lane_o == 2, sobj_b, 0.0)
        + jnp.where(lane_o == 3, sncorr_b, 0.0)
        + jnp.where(lane_o == 4, scoord_b, 0.0)
    )

    @pl.when(g == 0)
    def _zero():
        out_ref[...] = jnp.zeros_like(out_ref)

    out_ref[...] = out_ref[...] + delta


def kernel(predictions, bboxes):
    B, H, W, A, C = predictions.shape
    pred_t = jnp.transpose(predictions, (0, 3, 4, 1, 2))

    anchors = jnp.asarray(_ANCHORS, dtype=jnp.float32)
    cx, cy = bboxes[..., 0], bboxes[..., 1]
    w, h = bboxes[..., 2], bboxes[..., 3]
    valid = ~jnp.all(bboxes == 0.0, axis=-1)
    gx = jnp.clip(jnp.floor(cx * W).astype(jnp.int32), 0, W - 1)
    gy = jnp.clip(jnp.floor(cy * H).astype(jnp.int32), 0, H - 1)
    aw = anchors[:, 0][None, None, :]
    ah = anchors[:, 1][None, None, :]
    inter = jnp.minimum(w[..., None], aw) * jnp.minimum(h[..., None], ah)
    union = (w * h)[..., None] + aw * ah - inter
    iou = inter / (union + 1e-16)
    best = jnp.argmax(iou, axis=-1).astype(jnp.int32)

    idx = jnp.stack([best, gy], axis=1)
    ixc = jnp.stack([best, gx, gy, valid.astype(jnp.int32)], axis=-1)
    ixr = jnp.transpose(ixc, (0, 2, 1))

    grid_spec = pltpu.PrefetchScalarGridSpec(
        num_scalar_prefetch=1,
        grid=(B // _NB,),
        in_specs=[
            pl.BlockSpec((_NB, A, C, H, W), lambda b, s: (b, 0, 0, 0, 0)),
            pl.BlockSpec((B, _N, 4), lambda b, s: (0, 0, 0)),
            pl.BlockSpec((B, _N, 4), lambda b, s: (0, 0, 0)),
            pl.BlockSpec((B, 4, _N), lambda b, s: (0, 0, 0)),
        ],
        out_specs=pl.BlockSpec((1, 128), lambda b, s: (0, 0)),
        scratch_shapes=[pltpu.VMEM((_N, 5, _W), jnp.float32)],
    )
    acc = pl.pallas_call(
        _loss_kernel,
        grid_spec=grid_spec,
        out_shape=jax.ShapeDtypeStruct((1, 128), jnp.float32),
    )(idx, pred_t, bboxes, ixc, ixr)

    s0, n_pos = acc[0, 0], acc[0, 1]
    sobj, sncorr, scoord = acc[0, 2], acc[0, 3], acc[0, 4]
    n_neg = jnp.float32(B * H * W * A) - n_pos
    coord_loss = 5.0 * scoord
    obj_loss = 1.0 * sobj
    noobj_loss = 0.5 * (s0 - sncorr)
    coord_loss = jnp.where(n_pos > 0, coord_loss / n_pos, coord_loss)
    obj_loss = jnp.where(n_pos > 0, obj_loss / n_pos, obj_loss)
    noobj_loss = jnp.where(n_neg > 0, noobj_loss / n_neg, noobj_loss)
    cls_loss = jnp.asarray(0.0, jnp.float32)
    total_loss = coord_loss + obj_loss + noobj_loss + cls_loss
    return (total_loss, coord_loss, obj_loss, noobj_loss, cls_loss)


# speedup vs baseline: 6.1618x; 1.0144x over previous
import math

import jax
import jax.numpy as jnp
from jax.experimental import pallas as pl
from jax.experimental.pallas import tpu as pltpu

_B, _H, _W, _A, _C = 32, 56, 56, 9, 6
_N = 20
_NB = 8
_ANCHORS = [
    (s * math.sqrt(r) / 224.0, s / math.sqrt(r) / 224.0)
    for s in (32, 64, 128)
    for r in (0.5, 1.0, 2.0)
]


def _loss_kernel(idx_smem, pred_ref, bbc_ref, ixc_ref, ixr_ref, out_ref, scr):
    g = pl.program_id(0)
    f32 = jnp.float32

    x = pred_ref[:, :, 4, :, :]
    p_all = jax.nn.sigmoid(x)
    s0 = jnp.sum(-jnp.maximum(jnp.log(1.0 - p_all), -100.0))

    npos_b = jnp.float32(0.0)
    sobj_b = jnp.float32(0.0)
    sncorr_b = jnp.float32(0.0)
    scoord_b = jnp.float32(0.0)
    n_c = jax.lax.broadcasted_iota(jnp.int32, (_N, 1), 0)
    n_r = jax.lax.broadcasted_iota(jnp.int32, (1, _N), 1)
    lane = jax.lax.broadcasted_iota(jnp.int32, (_N, _W), 1)

    for i in range(_NB):
        b = g * _NB + i
        bbc = bbc_ref[b]
        cx, cy = bbc[:, 0:1], bbc[:, 1:2]
        w, h = bbc[:, 2:3], bbc[:, 3:4]
        ixc = ixc_ref[b]
        best_c, gx_c, gy_c = ixc[:, 0:1], ixc[:, 1:2], ixc[:, 2:3]
        valid_c = ixc[:, 3:4] != 0
        tx = cx * _W - gx_c.astype(f32)
        ty = cy * _H - gy_c.astype(f32)
        baw = jnp.zeros(best_c.shape, f32)
        bah = jnp.zeros(best_c.shape, f32)
        for k, (awk, ahk) in enumerate(_ANCHORS):
            baw = jnp.where(best_c == k, awk, baw)
            bah = jnp.where(best_c == k, ahk, bah)
        tw = jnp.log(w / baw + 1e-16)
        th = jnp.log(h / bah + 1e-16)
        flat_c = (gy_c * _W + gx_c) * _A + best_c

        ixr = ixr_ref[b]
        flat_r = (ixr[2:3, :] * _W + ixr[1:2, :]) * _A + ixr[0:1, :]
        valid_r = ixr[3:4, :] != 0

        id_c = jnp.where(valid_c, flat_c, -1 - n_c)
        id_r = jnp.where(valid_r, flat_r, -1 - n_r)
        killed = jnp.any((id_c == id_r) & (n_r > n_c), axis=1, keepdims=True)
        alive = (valid_c & ~killed).astype(f32)

        for n in range(_N):
            bestn = idx_smem[b, 0, n]
            gyn = idx_smem[b, 2, n]
            scr[n] = pred_ref[i, bestn, 0:5, gyn, :]
        sel = lane == gx_c
        v = [
            jnp.sum(jnp.where(sel, scr[:, c, :], 0.0), axis=1, keepdims=True)
            for c in range(5)
        ]
        coord_n = (
            (v[0] - tx) ** 2 + (v[1] - ty) ** 2 + (v[2] - tw) ** 2 + (v[3] - th) ** 2
        )
        pv = jax.nn.sigmoid(v[4])
        obj_n = -jnp.maximum(jnp.log(pv), -100.0)
        ncorr_n = -jnp.maximum(jnp.log(1.0 - pv), -100.0)

        npos_b += jnp.sum(alive)
        sobj_b += jnp.sum(alive * obj_n)
        sncorr_b += jnp.sum(alive * ncorr_n)
        scoord_b += jnp.sum(alive * coord_n)

    lane_o = jax.lax.broadcasted_iota(jnp.int32, (1, 128), 1)
    delta = (
        jnp.where(lane_o == 0, s0, 0.0)
        + jnp.where(lane_o == 1, npos_b, 0.0)
        + jnp.where(lane_o == 2, sobj_b, 0.0)
        + jnp.where(lane_o == 3, sncorr_b, 0.0)
        + jnp.where(lane_o == 4, scoord_b, 0.0)
    )

    @pl.when(g == 0)
    def _zero():
        out_ref[...] = jnp.zeros_like(out_ref)

    out_ref[...] = out_ref[...] + delta


def kernel(predictions, bboxes):
    B, H, W, A, C = predictions.shape
    pred_t = jnp.transpose(predictions, (0, 3, 4, 1, 2))

    anchors = jnp.asarray(_ANCHORS, dtype=jnp.float32)
    cx, cy = bboxes[..., 0], bboxes[..., 1]
    w, h = bboxes[..., 2], bboxes[..., 3]
    valid = ~jnp.all(bboxes == 0.0, axis=-1)
    gx = jnp.clip(jnp.floor(cx * W).astype(jnp.int32), 0, W - 1)
    gy = jnp.clip(jnp.floor(cy * H).astype(jnp.int32), 0, H - 1)
    aw = anchors[:, 0][None, None, :]
    ah = anchors[:, 1][None, None, :]
    inter = jnp.minimum(w[..., None], aw) * jnp.minimum(h[..., None], ah)
    union = (w * h)[..., None] + aw * ah - inter
    iou = inter / (union + 1e-16)
    best = jnp.argmax(iou, axis=-1).astype(jnp.int32)

    ixc = jnp.stack([best, gx, gy, valid.astype(jnp.int32)], axis=-1)
    ixr = jnp.transpose(ixc, (0, 2, 1))

    grid_spec = pltpu.PrefetchScalarGridSpec(
        num_scalar_prefetch=1,
        grid=(B // _NB,),
        in_specs=[
            pl.BlockSpec((_NB, A, 5, H, W), lambda b, s: (b, 0, 0, 0, 0)),
            pl.BlockSpec((B, _N, 4), lambda b, s: (0, 0, 0)),
            pl.BlockSpec((B, _N, 4), lambda b, s: (0, 0, 0)),
            pl.BlockSpec((B, 4, _N), lambda b, s: (0, 0, 0)),
        ],
        out_specs=pl.BlockSpec((1, 128), lambda b, s: (0, 0)),
        scratch_shapes=[pltpu.VMEM((_N, 5, _W), jnp.float32)],
    )
    acc = pl.pallas_call(
        _loss_kernel,
        grid_spec=grid_spec,
        out_shape=jax.ShapeDtypeStruct((1, 128), jnp.float32),
    )(ixr, pred_t, bboxes, ixc, ixr)

    s0, n_pos = acc[0, 0], acc[0, 1]
    sobj, sncorr, scoord = acc[0, 2], acc[0, 3], acc[0, 4]
    n_neg = jnp.float32(B * H * W * A) - n_pos
    coord_loss = 5.0 * scoord
    obj_loss = 1.0 * sobj
    noobj_loss = 0.5 * (s0 - sncorr)
    coord_loss = jnp.where(n_pos > 0, coord_loss / n_pos, coord_loss)
    obj_loss = jnp.where(n_pos > 0, obj_loss / n_pos, obj_loss)
    noobj_loss = jnp.where(n_neg > 0, noobj_loss / n_neg, noobj_loss)
    cls_loss = jnp.asarray(0.0, jnp.float32)
    total_loss = coord_loss + obj_loss + noobj_loss + cls_loss
    return (total_loss, coord_loss, obj_loss, noobj_loss, cls_loss)


# speedup vs baseline: 6.6339x; 1.0766x over previous
import math

import jax
import jax.numpy as jnp
from jax.experimental import pallas as pl
from jax.experimental.pallas import tpu as pltpu

_B, _H, _W, _A, _C = 32, 56, 56, 9, 6
_N = 20
_NB = 8
_ANCHORS = [
    (s * math.sqrt(r) / 224.0, s / math.sqrt(r) / 224.0)
    for s in (32, 64, 128)
    for r in (0.5, 1.0, 2.0)
]


def _loss_kernel(idx_smem, pred_ref, bbc_ref, ixc_ref, out_ref, scr):
    g = pl.program_id(0)
    f32 = jnp.float32

    x = pred_ref[:, :, 4, :, :]
    p_all = jax.nn.sigmoid(x)
    s0 = jnp.sum(-jnp.maximum(jnp.log(1.0 - p_all), -100.0))

    npos_b = jnp.float32(0.0)
    sobj_b = jnp.float32(0.0)
    sncorr_b = jnp.float32(0.0)
    scoord_b = jnp.float32(0.0)
    n_c = jax.lax.broadcasted_iota(jnp.int32, (_N, 1), 0)
    n_r = jax.lax.broadcasted_iota(jnp.int32, (1, _N), 1)
    lane = jax.lax.broadcasted_iota(jnp.int32, (_N, _W), 1)

    for i in range(_NB):
        b = g * _NB + i
        bbc = bbc_ref[b]
        cx, cy = bbc[:, 0:1], bbc[:, 1:2]
        w, h = bbc[:, 2:3], bbc[:, 3:4]
        ixc = ixc_ref[b]
        best_c, gx_c, gy_c = ixc[:, 0:1], ixc[:, 1:2], ixc[:, 2:3]
        valid_c = ixc[:, 3:4] != 0
        tx = cx * _W - gx_c.astype(f32)
        ty = cy * _H - gy_c.astype(f32)
        baw = jnp.zeros(best_c.shape, f32)
        bah = jnp.zeros(best_c.shape, f32)
        for k, (awk, ahk) in enumerate(_ANCHORS):
            baw = jnp.where(best_c == k, awk, baw)
            bah = jnp.where(best_c == k, ahk, bah)
        tw = jnp.log(w / baw + 1e-16)
        th = jnp.log(h / bah + 1e-16)
        flat_c = (gy_c * _W + gx_c) * _A + best_c

        id_c = jnp.where(valid_c, flat_c, -1 - n_c)
        id_r = jnp.transpose(id_c, (1, 0))
        killed = jnp.any((id_c == id_r) & (n_r > n_c), axis=1, keepdims=True)
        alive = (valid_c & ~killed).astype(f32)

        for n in range(_N):
            bestn = idx_smem[b, n, 0]
            gyn = idx_smem[b, n, 2]
            scr[n] = pred_ref[i, bestn, 0:5, gyn, :]
        sel = lane == gx_c
        v = [
            jnp.sum(jnp.where(sel, scr[:, c, :], 0.0), axis=1, keepdims=True)
            for c in range(5)
        ]
        coord_n = (
            (v[0] - tx) ** 2 + (v[1] - ty) ** 2 + (v[2] - tw) ** 2 + (v[3] - th) ** 2
        )
        pv = jax.nn.sigmoid(v[4])
        obj_n = -jnp.maximum(jnp.log(pv), -100.0)
        ncorr_n = -jnp.maximum(jnp.log(1.0 - pv), -100.0)

        npos_b += jnp.sum(alive)
        sobj_b += jnp.sum(alive * obj_n)
        sncorr_b += jnp.sum(alive * ncorr_n)
        scoord_b += jnp.sum(alive * coord_n)

    lane_o = jax.lax.broadcasted_iota(jnp.int32, (1, 128), 1)
    delta = (
        jnp.where(lane_o == 0, s0, 0.0)
        + jnp.where(lane_o == 1, npos_b, 0.0)
        + jnp.where(lane_o == 2, sobj_b, 0.0)
        + jnp.where(lane_o == 3, sncorr_b, 0.0)
        + jnp.where(lane_o == 4, scoord_b, 0.0)
    )

    @pl.when(g == 0)
    def _zero():
        out_ref[...] = jnp.zeros_like(out_ref)

    out_ref[...] = out_ref[...] + delta

    @pl.when(g == _B // _NB - 1)
    def _finalize():
        a = out_ref[...]
        s0s = jnp.sum(jnp.where(lane_o == 0, a, 0.0))
        npos = jnp.sum(jnp.where(lane_o == 1, a, 0.0))
        sobj = jnp.sum(jnp.where(lane_o == 2, a, 0.0))
        sncorr = jnp.sum(jnp.where(lane_o == 3, a, 0.0))
        scoord = jnp.sum(jnp.where(lane_o == 4, a, 0.0))
        n_neg = jnp.float32(_B * _H * _W * _A) - npos
        coord_loss = 5.0 * scoord
        noobj_loss = 0.5 * (s0s - sncorr)
        coord_loss = jnp.where(npos > 0, coord_loss / npos, coord_loss)
        obj_loss = jnp.where(npos > 0, sobj / npos, sobj)
        noobj_loss = jnp.where(n_neg > 0, noobj_loss / n_neg, noobj_loss)
        total_loss = coord_loss + obj_loss + noobj_loss
        out_ref[...] = (
            jnp.where(lane_o == 0, total_loss, 0.0)
            + jnp.where(lane_o == 1, coord_loss, 0.0)
            + jnp.where(lane_o == 2, obj_loss, 0.0)
            + jnp.where(lane_o == 3, noobj_loss, 0.0)
        )


def kernel(predictions, bboxes):
    B, H, W, A, C = predictions.shape
    pred_t = jnp.transpose(predictions, (0, 3, 4, 1, 2))

    anchors = jnp.asarray(_ANCHORS, dtype=jnp.float32)
    cx, cy = bboxes[..., 0], bboxes[..., 1]
    w, h = bboxes[..., 2], bboxes[..., 3]
    valid = ~jnp.all(bboxes == 0.0, axis=-1)
    gx = jnp.clip(jnp.floor(cx * W).astype(jnp.int32), 0, W - 1)
    gy = jnp.clip(jnp.floor(cy * H).astype(jnp.int32), 0, H - 1)
    aw = anchors[:, 0][None, None, :]
    ah = anchors[:, 1][None, None, :]
    inter = jnp.minimum(w[..., None], aw) * jnp.minimum(h[..., None], ah)
    union = (w * h)[..., None] + aw * ah - inter
    iou = inter / (union + 1e-16)
    best = jnp.argmax(iou, axis=-1).astype(jnp.int32)

    ixc = jnp.stack([best, gx, gy, valid.astype(jnp.int32)], axis=-1)

    grid_spec = pltpu.PrefetchScalarGridSpec(
        num_scalar_prefetch=1,
        grid=(B // _NB,),
        in_specs=[
            pl.BlockSpec((_NB, A, 5, H, W), lambda b, s: (b, 0, 0, 0, 0)),
            pl.BlockSpec((B, _N, 4), lambda b, s: (0, 0, 0)),
            pl.BlockSpec((B, _N, 4), lambda b, s: (0, 0, 0)),
        ],
        out_specs=pl.BlockSpec((1, 128), lambda b, s: (0, 0)),
        scratch_shapes=[pltpu.VMEM((_N, 5, _W), jnp.float32)],
    )
    acc = pl.pallas_call(
        _loss_kernel,
        grid_spec=grid_spec,
        out_shape=jax.ShapeDtypeStruct((1, 128), jnp.float32),
    )(ixc, pred_t, bboxes, ixc)

    cls_loss = jnp.asarray(0.0, jnp.float32)
    return (acc[0, 0], acc[0, 1], acc[0, 2], acc[0, 3], cls_loss)


# speedup vs baseline: 8.3048x; 1.2519x over previous
import math

import jax
import jax.numpy as jnp
from jax.experimental import pallas as pl
from jax.experimental.pallas import tpu as pltpu

_B, _H, _W, _A, _C = 32, 56, 56, 9, 6
_N = 20
_NB = 8
_ANCHORS = [
    (s * math.sqrt(r) / 224.0, s / math.sqrt(r) / 224.0)
    for s in (32, 64, 128)
    for r in (0.5, 1.0, 2.0)
]


def _loss_kernel(idx_smem, pred_ref, bbc_ref, ixc_ref, out_ref, scr):
    g = pl.program_id(0)
    f32 = jnp.float32

    x = pred_ref[:, :, 4, :, :]
    p_all = jax.nn.sigmoid(x)
    s0 = jnp.sum(-jnp.maximum(jnp.log(1.0 - p_all), -100.0))

    npos_b = jnp.float32(0.0)
    sobj_b = jnp.float32(0.0)
    sncorr_b = jnp.float32(0.0)
    scoord_b = jnp.float32(0.0)
    n_c = jax.lax.broadcasted_iota(jnp.int32, (_N, 1), 0)
    n_r = jax.lax.broadcasted_iota(jnp.int32, (1, _N), 1)
    lane = jax.lax.broadcasted_iota(jnp.int32, (_N, _W), 1)

    for i in range(_NB):
        b = g * _NB + i
        bbc = bbc_ref[b]
        cx, cy = bbc[:, 0:1], bbc[:, 1:2]
        w, h = bbc[:, 2:3], bbc[:, 3:4]
        pk = ixc_ref[b]
        valid_c = (pk & 1) != 0
        best_c = (pk >> 1) & 15
        gx_c = (pk >> 5) & 63
        gy_c = (pk >> 11) & 63
        tx = cx * _W - gx_c.astype(f32)
        ty = cy * _H - gy_c.astype(f32)
        baw = jnp.zeros(best_c.shape, f32)
        bah = jnp.zeros(best_c.shape, f32)
        for k, (awk, ahk) in enumerate(_ANCHORS):
            baw = jnp.where(best_c == k, awk, baw)
            bah = jnp.where(best_c == k, ahk, bah)
        tw = jnp.log(w / baw + 1e-16)
        th = jnp.log(h / bah + 1e-16)
        flat_c = (gy_c * _W + gx_c) * _A + best_c

        id_c = jnp.where(valid_c, flat_c, -1 - n_c)
        id_r = jnp.transpose(id_c, (1, 0))
        killed = jnp.any((id_c == id_r) & (n_r > n_c), axis=1, keepdims=True)
        alive = (valid_c & ~killed).astype(f32)

        for n in range(_N):
            s = idx_smem[b, n, 0]
            bestn = (s >> 1) & 15
            gyn = (s >> 11) & 63
            scr[n] = pred_ref[i, bestn, 0:5, gyn, :]
        sel = lane == gx_c
        v = [
            jnp.sum(jnp.where(sel, scr[:, c, :], 0.0), axis=1, keepdims=True)
            for c in range(5)
        ]
        coord_n = (
            (v[0] - tx) ** 2 + (v[1] - ty) ** 2 + (v[2] - tw) ** 2 + (v[3] - th) ** 2
        )
        pv = jax.nn.sigmoid(v[4])
        obj_n = -jnp.maximum(jnp.log(pv), -100.0)
        ncorr_n = -jnp.maximum(jnp.log(1.0 - pv), -100.0)

        npos_b += jnp.sum(alive)
        sobj_b += jnp.sum(alive * obj_n)
        sncorr_b += jnp.sum(alive * ncorr_n)
        scoord_b += jnp.sum(alive * coord_n)

    lane_o = jax.lax.broadcasted_iota(jnp.int32, (1, 128), 1)
    delta = (
        jnp.where(lane_o == 0, s0, 0.0)
        + jnp.where(lane_o == 1, npos_b, 0.0)
        + jnp.where(lane_o == 2, sobj_b, 0.0)
        + jnp.where(lane_o == 3, sncorr_b, 0.0)
        + jnp.where(lane_o == 4, scoord_b, 0.0)
    )

    @pl.when(g == 0)
    def _zero():
        out_ref[...] = jnp.zeros_like(out_ref)

    out_ref[...] = out_ref[...] + delta

    @pl.when(g == _B // _NB - 1)
    def _finalize():
        a = out_ref[...]
        s0s = jnp.sum(jnp.where(lane_o == 0, a, 0.0))
        npos = jnp.sum(jnp.where(lane_o == 1, a, 0.0))
        sobj = jnp.sum(jnp.where(lane_o == 2, a, 0.0))
        sncorr = jnp.sum(jnp.where(lane_o == 3, a, 0.0))
        scoord = jnp.sum(jnp.where(lane_o == 4, a, 0.0))
        n_neg = jnp.float32(_B * _H * _W * _A) - npos
        coord_loss = 5.0 * scoord
        noobj_loss = 0.5 * (s0s - sncorr)
        coord_loss = jnp.where(npos > 0, coord_loss / npos, coord_loss)
        obj_loss = jnp.where(npos > 0, sobj / npos, sobj)
        noobj_loss = jnp.where(n_neg > 0, noobj_loss / n_neg, noobj_loss)
        total_loss = coord_loss + obj_loss + noobj_loss
        out_ref[...] = (
            jnp.where(lane_o == 0, total_loss, 0.0)
            + jnp.where(lane_o == 1, coord_loss, 0.0)
            + jnp.where(lane_o == 2, obj_loss, 0.0)
            + jnp.where(lane_o == 3, noobj_loss, 0.0)
        )


def kernel(predictions, bboxes):
    B, H, W, A, C = predictions.shape
    pred_t = jnp.transpose(predictions, (0, 3, 4, 1, 2))

    anchors = jnp.asarray(_ANCHORS, dtype=jnp.float32)
    cx, cy = bboxes[..., 0], bboxes[..., 1]
    w, h = bboxes[..., 2], bboxes[..., 3]
    valid = ~jnp.all(bboxes == 0.0, axis=-1)
    gx = jnp.clip(jnp.floor(cx * W).astype(jnp.int32), 0, W - 1)
    gy = jnp.clip(jnp.floor(cy * H).astype(jnp.int32), 0, H - 1)
    aw = anchors[:, 0][None, None, :]
    ah = anchors[:, 1][None, None, :]
    inter = jnp.minimum(w[..., None], aw) * jnp.minimum(h[..., None], ah)
    union = (w * h)[..., None] + aw * ah - inter
    iou = inter / (union + 1e-16)
    best = jnp.argmax(iou, axis=-1).astype(jnp.int32)

    pk = (((gy * 64 + gx) * 16 + best) * 2 + valid.astype(jnp.int32)).reshape(
        B, _N, 1
    )

    grid_spec = pltpu.PrefetchScalarGridSpec(
        num_scalar_prefetch=1,
        grid=(B // _NB,),
        in_specs=[
            pl.BlockSpec((_NB, A, 5, H, W), lambda b, s: (b, 0, 0, 0, 0)),
            pl.BlockSpec((B, _N, 4), lambda b, s: (0, 0, 0)),
            pl.BlockSpec((B, _N, 1), lambda b, s: (0, 0, 0)),
        ],
        out_specs=pl.BlockSpec((1, 128), lambda b, s: (0, 0)),
        scratch_shapes=[pltpu.VMEM((_N, 5, _W), jnp.float32)],
    )
    acc = pl.pallas_call(
        _loss_kernel,
        grid_spec=grid_spec,
        out_shape=jax.ShapeDtypeStruct((1, 128), jnp.float32),
    )(pk, pred_t, bboxes, pk)

    cls_loss = jnp.asarray(0.0, jnp.float32)
    return (acc[0, 0], acc[0, 1], acc[0, 2], acc[0, 3], cls_loss)
